# baseline (speedup 1.0000x reference)
import jax
import jax.numpy as jnp
from jax.experimental import pallas as pl
from jax.experimental.pallas import tpu as pltpu

_TN = 256


def _make_kernel(nt, nsteps):
    def _tile_kernel(s_ref, sb_ref, tb_ref, t2h_ref, thl_ref, out_ref,
                     gs_ref, ss_ref, acc_ref):
        k = pl.program_id(0)
        s = s_ref[0]
        prod = jax.lax.dot_general(
            sb_ref[0], tb_ref[0], (((1,), (0,)), ((), ())),
            preferred_element_type=jnp.float32,
        )
        q = t2h_ref[0] - 2.0 * prod
        rowmin = jnp.min(q, axis=1)
        onehot = jnp.where(q == rowmin[:, None], 1.0, 0.0).astype(jnp.bfloat16)
        g = jax.lax.dot_general(
            onehot, thl_ref[0], (((1,), (0,)), ((), ())),
            preferred_element_type=jnp.float32,
        )

        def _contrib(gv, sv):
            c = gv[:, 6:7]
            num = c * sv - (gv[:, 0:3] + gv[:, 3:6])
            return num * num / (c * c)

        @pl.when(k == 0)
        def _init():
            acc_ref[...] = jnp.zeros(acc_ref.shape, jnp.float32)

        @pl.when(k > 0)
        def _tail_prev():
            acc_ref[...] += _contrib(gs_ref[...], ss_ref[...])

        gs_ref[...] = g
        ss_ref[...] = s

        @pl.when(k == nsteps - 1)
        def _final():
            tot = jnp.sum(acc_ref[...] + _contrib(g, s))
            out_ref[...] = jnp.full((8, 128), tot, jnp.float32)

    return _tile_kernel


def _bf16_hi(x):
    return x.astype(jnp.bfloat16).astype(jnp.float32)


def kernel(source_point_cloud, target_point_cloud):
    B, N, _ = source_point_cloud.shape
    M = target_point_cloud.shape[1]
    nt = N // _TN
    bf16 = jnp.bfloat16

    src = source_point_cloud
    tgt = target_point_cloud

    s_bf = src.astype(bf16)
    t_bf = jnp.transpose(tgt, (0, 2, 1)).astype(bf16)
    t2h = jnp.sum(tgt * tgt, axis=2)[:, None, :]

    th = _bf16_hi(tgt)
    thl = jnp.concatenate(
        [th.astype(bf16), (tgt - th).astype(bf16), jnp.ones((B, M, 1), bf16)],
        axis=2,
    )

    out = pl.pallas_call(
        _make_kernel(nt, B * nt),
        grid=(B * nt,),
        in_specs=[
            pl.BlockSpec((1, _TN, 3), lambda k: (k // nt, k % nt, 0)),
            pl.BlockSpec((1, _TN, 3), lambda k: (k // nt, k % nt, 0)),
            pl.BlockSpec((1, 3, M), lambda k: (k // nt, 0, 0)),
            pl.BlockSpec((1, 1, M), lambda k: (k // nt, 0, 0)),
            pl.BlockSpec((1, M, 7), lambda k: (k // nt, 0, 0)),
        ],
        out_specs=pl.BlockSpec((8, 128), lambda k: (0, 0)),
        out_shape=jax.ShapeDtypeStruct((8, 128), jnp.float32),
        scratch_shapes=[
            pltpu.VMEM((_TN, 7), jnp.float32),
            pltpu.VMEM((_TN, 3), jnp.float32),
            pltpu.VMEM((_TN, 3), jnp.float32),
        ],
        compiler_params=pltpu.CompilerParams(
            dimension_semantics=("arbitrary",),
        ),
    )(src, s_bf, t_bf, t2h, thl)

    return out[0, 0] / (B * N * 3)


# speedup vs baseline: 1.0053x; 1.0053x over previous
import jax
import jax.numpy as jnp
from jax.experimental import pallas as pl
from jax.experimental.pallas import tpu as pltpu

_TN = 256


def _make_kernel(nt, nsteps):
    def _tile_kernel(s_ref, sb_ref, tb_ref, t2h_ref, thl_ref, out_ref,
                     gs_ref, ss_ref, acc_ref):
        k = pl.program_id(0) * nt + pl.program_id(1)
        s = s_ref[0]
        prod = jax.lax.dot_general(
            sb_ref[0], tb_ref[0], (((1,), (0,)), ((), ())),
            preferred_element_type=jnp.float32,
        )
        q = t2h_ref[0] - 2.0 * prod
        rowmin = jnp.min(q, axis=1)
        onehot = jnp.where(q == rowmin[:, None], 1.0, 0.0).astype(jnp.bfloat16)
        g = jax.lax.dot_general(
            onehot, thl_ref[0], (((1,), (0,)), ((), ())),
            preferred_element_type=jnp.float32,
        )

        def _contrib(gv, sv):
            c = gv[:, 6:7]
            num = c * sv - (gv[:, 0:3] + gv[:, 3:6])
            return num * num / (c * c)

        @pl.when(k == 0)
        def _init():
            acc_ref[...] = jnp.zeros(acc_ref.shape, jnp.float32)

        @pl.when(k > 0)
        def _tail_prev():
            acc_ref[...] += _contrib(gs_ref[...], ss_ref[...])

        gs_ref[...] = g
        ss_ref[...] = s

        @pl.when(k == nsteps - 1)
        def _final():
            tot = jnp.sum(acc_ref[...] + _contrib(g, s))
            out_ref[...] = jnp.full((8, 128), tot, jnp.float32)

    return _tile_kernel


def _bf16_hi(x):
    return x.astype(jnp.bfloat16).astype(jnp.float32)


def kernel(source_point_cloud, target_point_cloud):
    B, N, _ = source_point_cloud.shape
    M = target_point_cloud.shape[1]
    nt = N // _TN
    bf16 = jnp.bfloat16

    src = source_point_cloud
    tgt = target_point_cloud

    s_bf = src.astype(bf16)
    t_bf = jnp.transpose(tgt, (0, 2, 1)).astype(bf16)
    t2h = jnp.sum(tgt * tgt, axis=2)[:, None, :]

    th = _bf16_hi(tgt)
    thl = jnp.concatenate(
        [th.astype(bf16), (tgt - th).astype(bf16), jnp.ones((B, M, 1), bf16)],
        axis=2,
    )

    out = pl.pallas_call(
        _make_kernel(nt, B * nt),
        grid=(B, nt),
        in_specs=[
            pl.BlockSpec((1, _TN, 3), lambda b, i: (b, i, 0)),
            pl.BlockSpec((1, _TN, 3), lambda b, i: (b, i, 0)),
            pl.BlockSpec((1, 3, M), lambda b, i: (b, 0, 0)),
            pl.BlockSpec((1, 1, M), lambda b, i: (b, 0, 0)),
            pl.BlockSpec((1, M, 7), lambda b, i: (b, 0, 0)),
        ],
        out_specs=pl.BlockSpec((8, 128), lambda b, i: (0, 0)),
        out_shape=jax.ShapeDtypeStruct((8, 128), jnp.float32),
        scratch_shapes=[
            pltpu.VMEM((_TN, 7), jnp.float32),
            pltpu.VMEM((_TN, 3), jnp.float32),
            pltpu.VMEM((_TN, 3), jnp.float32),
        ],
        compiler_params=pltpu.CompilerParams(
            dimension_semantics=("arbitrary", "arbitrary"),
        ),
    )(src, s_bf, t_bf, t2h, thl)

    return out[0, 0] / (B * N * 3)
